# Initial kernel scaffold; baseline (speedup 1.0000x reference)
import jax
import jax.numpy as jnp
from jax.experimental import pallas as pl


def kernel(x, source_edge_index, sink_edge_index, W_src_init, b_src_init, W_snk_init, b_snk_init, W_layers_src, b_layers_src, W_layers_snk, b_layers_snk, W_src_out, b_src_out, W_snk_out, b_snk_out, W_dim1_out, b_dim1_out):
    raise NotImplementedError("write your pallas kernel here")


# speedup vs baseline: 5.2165x; 5.2165x over previous
import functools

import jax
import jax.numpy as jnp
from jax import lax
from jax.experimental import pallas as pl
from jax.experimental.pallas import tpu as pltpu
from jax.experimental.pallas import tpu_sc as plsc

_N = 100000
_E = 1600000
_H = 32
_HH = _H // 2
_TILES = 16
_B = 128
_PER_TILE = _E // _TILES
_NB = _PER_TILE // _B
_TAIL = _PER_TILE - _NB * _B
_RPT = 6248
_RTAIL = _N - _RPT * _TILES
_ZCH = 568


_sc_mesh = plsc.VectorSubcoreMesh(core_axis_name="c", subcore_axis_name="s")


@functools.partial(
    pl.kernel,
    out_type=jax.ShapeDtypeStruct((2 * _N, _HH), jnp.float32),
    mesh=_sc_mesh,
    scratch_types=[
        pltpu.VMEM_SHARED((_N, _HH), jnp.float32),
        pltpu.VMEM((_B,), jnp.int32),
        pltpu.VMEM((_B,), jnp.int32),
        pltpu.VMEM((_B,), jnp.int32),
        pltpu.VMEM((_B, _HH), jnp.float32),
        pltpu.VMEM((_TAIL,), jnp.int32),
        pltpu.VMEM((_TAIL,), jnp.int32),
        pltpu.VMEM((_TAIL,), jnp.int32),
        pltpu.VMEM((_TAIL, _HH), jnp.float32),
        pltpu.VMEM((_ZCH, _HH), jnp.float32),
        pltpu.SemaphoreType.DMA,
    ],
    compiler_params=pltpu.CompilerParams(use_tc_tiling_on_sc=False),
)
def _sc_segsum(table, src, dst, out, accum, sidx, didx, gidx, rows,
               sidx2, didx2, gidx2, rows2, zbuf, sem):
    c = lax.axis_index("c")
    s = lax.axis_index("s")

    def _zrow(i, _):
        zbuf[i, :] = jnp.zeros((_HH,), jnp.float32)
        return None
    lax.fori_loop(0, _ZCH, _zrow, None)

    def _zchunk(k, _):
        pltpu.sync_copy(zbuf, accum.at[pl.ds(s * _RPT + k * _ZCH, _ZCH)])
        return None
    lax.fori_loop(0, _RPT // _ZCH, _zchunk, None)

    @pl.when(s == 0)
    def _ztail():
        pltpu.sync_copy(zbuf.at[pl.ds(0, _RTAIL)],
                        accum.at[pl.ds(_RPT * _TILES, _RTAIL)])
    plsc.subcore_barrier()

    def _batch(j, _):
        base = s * _PER_TILE + j * _B
        pltpu.sync_copy(src.at[pl.ds(base, _B)], sidx)
        pltpu.sync_copy(dst.at[pl.ds(base, _B)], didx)

        def _g(i, _):
            gidx[pl.ds(i * 16, 16)] = sidx[pl.ds(i * 16, 16)] * 2 + c
            return None
        lax.fori_loop(0, _B // 16, _g, None)

        pltpu.async_copy(table.at[gidx], rows, sem).wait()
        pltpu.sync_copy(rows, accum.at[didx], add=True)
        return None
    lax.fori_loop(0, _NB, _batch, None)

    base_t = s * _PER_TILE + _NB * _B
    pltpu.sync_copy(src.at[pl.ds(base_t, _TAIL)], sidx2)
    pltpu.sync_copy(dst.at[pl.ds(base_t, _TAIL)], didx2)
    for i in range(_TAIL // 16):
        gidx2[pl.ds(i * 16, 16)] = sidx2[pl.ds(i * 16, 16)] * 2 + c
    pltpu.async_copy(table.at[gidx2], rows2, sem).wait()
    pltpu.sync_copy(rows2, accum.at[didx2], add=True)

    plsc.subcore_barrier()

    r0 = s * _RPT
    pltpu.sync_copy(accum.at[pl.ds(r0, _RPT)],
                    out.at[pl.ds(c * _N + r0, _RPT)])

    @pl.when(s == 0)
    def _rtail():
        pltpu.sync_copy(accum.at[pl.ds(_RPT * _TILES, _RTAIL)],
                        out.at[pl.ds(c * _N + _RPT * _TILES, _RTAIL)])


_BLK = 2000
_G = _N // _BLK

_row = lambda i: (i, 0)
_zero = lambda i: (0, 0)
_lo = lambda i: (i, 0)
_hi = lambda i: (_G + i, 0)


def _full(shape):
    return pl.BlockSpec(shape, _zero)


def _tc_init_body(x_ref, ws_ref, bs_ref, wk_ref, bk_ref, se_ref, ke_ref):
    xb = x_ref[...]
    se_ref[...] = xb * ws_ref[...] + bs_ref[...]
    ke_ref[...] = xb * wk_ref[...] + bk_ref[...]


_tc_init = pl.pallas_call(
    _tc_init_body,
    grid=(_G,),
    in_specs=[
        pl.BlockSpec((_BLK, 1), _row),
        _full((1, _H)), _full((1, _H)), _full((1, _H)), _full((1, _H)),
    ],
    out_specs=[pl.BlockSpec((_BLK, _H), _row)] * 2,
    out_shape=[jax.ShapeDtypeStruct((_N, _H), jnp.float32)] * 2,
)


def _tc_layer_body(alo_ref, ahi_ref, e_ref, w_ref, b_ref, o_ref):
    a = jnp.concatenate([alo_ref[...], ahi_ref[...]], axis=1)
    t = a + e_ref[...]
    o_ref[...] = jnp.maximum(
        jnp.dot(t, w_ref[...], preferred_element_type=jnp.float32)
        + b_ref[...], 0.0)


_tc_layer = pl.pallas_call(
    _tc_layer_body,
    grid=(_G,),
    in_specs=[
        pl.BlockSpec((_BLK, _HH), _lo),
        pl.BlockSpec((_BLK, _HH), _hi),
        pl.BlockSpec((_BLK, _H), _row),
        _full((_H, _H)), _full((1, _H)),
    ],
    out_specs=pl.BlockSpec((_BLK, _H), _row),
    out_shape=jax.ShapeDtypeStruct((_N, _H), jnp.float32),
)


def _tc_comb_body(se_ref, ke_ref, x_ref, wso_ref, bso_ref, wko_ref, bko_ref,
                  pd0_ref, v_ref):
    comb = se_ref[...] + ke_ref[...]
    so = jnp.maximum(
        jnp.dot(comb, wso_ref[...], preferred_element_type=jnp.float32)
        + bso_ref[...], 0.0)
    ko = jnp.maximum(
        jnp.dot(comb, wko_ref[...], preferred_element_type=jnp.float32)
        + bko_ref[...], 0.0)
    pd0_ref[...] = (so + ko) * 0.5
    v_ref[...] = comb * x_ref[...]


_tc_comb = pl.pallas_call(
    _tc_comb_body,
    grid=(_G,),
    in_specs=[
        pl.BlockSpec((_BLK, _H), _row),
        pl.BlockSpec((_BLK, _H), _row),
        pl.BlockSpec((_BLK, 1), _row),
        _full((_H, 2)), _full((1, 2)), _full((_H, 2)), _full((1, 2)),
    ],
    out_specs=[
        pl.BlockSpec((_BLK, 2), _row),
        pl.BlockSpec((_BLK, _H), _row),
    ],
    out_shape=[
        jax.ShapeDtypeStruct((_N, 2), jnp.float32),
        jax.ShapeDtypeStruct((_N, _H), jnp.float32),
    ],
)


def _tc_final_body(aslo_ref, ashi_ref, aklo_ref, akhi_ref, pd0_ref,
                   wd_ref, bd_ref, o_ref):
    s1 = jnp.maximum(
        jnp.concatenate([aslo_ref[...], ashi_ref[...]], axis=1), 0.0)
    k1 = jnp.maximum(
        jnp.concatenate([aklo_ref[...], akhi_ref[...]], axis=1), 0.0)
    pd1 = jnp.maximum(
        jnp.dot(s1 + k1, wd_ref[...], preferred_element_type=jnp.float32)
        + bd_ref[...], 0.0)
    o_ref[...] = jnp.concatenate([pd0_ref[...], pd1], axis=1)


_tc_final = pl.pallas_call(
    _tc_final_body,
    grid=(_G,),
    in_specs=[
        pl.BlockSpec((_BLK, _HH), _lo),
        pl.BlockSpec((_BLK, _HH), _hi),
        pl.BlockSpec((_BLK, _HH), _lo),
        pl.BlockSpec((_BLK, _HH), _hi),
        pl.BlockSpec((_BLK, 2), _row),
        _full((_H, 2)), _full((1, 2)),
    ],
    out_specs=pl.BlockSpec((_BLK, 4), _row),
    out_shape=jax.ShapeDtypeStruct((_N, 4), jnp.float32),
)


def kernel(x, source_edge_index, sink_edge_index, W_src_init, b_src_init,
           W_snk_init, b_snk_init, W_layers_src, b_layers_src, W_layers_snk,
           b_layers_snk, W_src_out, b_src_out, W_snk_out, b_snk_out,
           W_dim1_out, b_dim1_out):
    ssrc, sdst = source_edge_index[0], source_edge_index[1]
    ksrc, kdst = sink_edge_index[0], sink_edge_index[1]

    se, ke = _tc_init(x, W_src_init, b_src_init.reshape(1, _H),
                      W_snk_init, b_snk_init.reshape(1, _H))

    for i in range(W_layers_src.shape[0]):
        aggs = _sc_segsum(se.reshape(2 * _N, _HH), ssrc, sdst)
        se = _tc_layer(aggs, aggs, se, W_layers_src[i],
                       b_layers_src[i].reshape(1, _H))
        aggk = _sc_segsum(ke.reshape(2 * _N, _HH), ksrc, kdst)
        ke = _tc_layer(aggk, aggk, ke, W_layers_snk[i],
                       b_layers_snk[i].reshape(1, _H))

    pd0, v = _tc_comb(se, ke, x, W_src_out, b_src_out.reshape(1, 2),
                      W_snk_out, b_snk_out.reshape(1, 2))

    v2 = v.reshape(2 * _N, _HH)
    a_s = _sc_segsum(v2, ssrc, sdst)
    a_k = _sc_segsum(v2, ksrc, kdst)

    return _tc_final(a_s, a_s, a_k, a_k, pd0, W_dim1_out,
                     b_dim1_out.reshape(1, 2))


# speedup vs baseline: 16.1310x; 3.0923x over previous
import functools

import jax
import jax.numpy as jnp
from jax import lax
from jax.experimental import pallas as pl
from jax.experimental.pallas import tpu as pltpu
from jax.experimental.pallas import tpu_sc as plsc

_N = 100000
_E = 1600000
_H = 32
_HH = _H // 2
_TILES = 16
_B = 128
_PER_TILE = _E // _TILES
_NB = _PER_TILE // _B
_TAIL = _PER_TILE - _NB * _B
_RPT = 6248
_RTAIL = _N - _RPT * _TILES
_ZCH = 568


_sc_mesh = plsc.VectorSubcoreMesh(core_axis_name="c", subcore_axis_name="s")


_R = 3
_NB_MAIN = _NB - 1


@functools.partial(
    pl.kernel,
    out_type=jax.ShapeDtypeStruct((2 * _N, _HH), jnp.float32),
    mesh=_sc_mesh,
    scratch_types=[
        pltpu.VMEM_SHARED((_N, _HH), jnp.float32),
        pltpu.VMEM((_R, _B), jnp.int32),
        pltpu.VMEM((_R, _B), jnp.int32),
        pltpu.VMEM((_R, _B, _HH), jnp.float32),
        pltpu.VMEM((_TAIL,), jnp.int32),
        pltpu.VMEM((_TAIL,), jnp.int32),
        pltpu.VMEM((_TAIL, _HH), jnp.float32),
        pltpu.VMEM((_ZCH, _HH), jnp.float32),
        [pltpu.SemaphoreType.DMA] * _R,
        [pltpu.SemaphoreType.DMA] * _R,
        pltpu.SemaphoreType.DMA,
    ],
    compiler_params=pltpu.CompilerParams(use_tc_tiling_on_sc=False),
)
def _sc_segsum(gsrc, dst, table, out, accum, gidx, didx, rows,
               gidx2, didx2, rows2, zbuf, gsems, isems, tsem):
    c = lax.axis_index("c")
    s = lax.axis_index("s")

    def _zrow(i, _):
        zbuf[i, :] = jnp.zeros((_HH,), jnp.float32)
        return None
    lax.fori_loop(0, _ZCH, _zrow, None)

    def _zchunk(k, _):
        pltpu.sync_copy(zbuf, accum.at[pl.ds(s * _RPT + k * _ZCH, _ZCH)])
        return None
    lax.fori_loop(0, _RPT // _ZCH, _zchunk, None)

    @pl.when(s == 0)
    def _ztail():
        pltpu.sync_copy(zbuf.at[pl.ds(0, _RTAIL)],
                        accum.at[pl.ds(_RPT * _TILES, _RTAIL)])
    plsc.subcore_barrier()

    gbase0 = c * _E + s * _PER_TILE
    dbase0 = s * _PER_TILE

    def _issue_idx(j, slot):
        gb = gbase0 + j * _B
        db = dbase0 + j * _B
        pltpu.async_copy(gsrc.at[pl.ds(gb, _B)], gidx.at[slot], isems[slot])
        pltpu.async_copy(dst.at[pl.ds(db, _B)], didx.at[slot], isems[slot])

    def _wait_idx(slot):
        pltpu.make_async_copy(gsrc.at[pl.ds(0, _B)], gidx.at[slot],
                              isems[slot]).wait()
        pltpu.make_async_copy(dst.at[pl.ds(0, _B)], didx.at[slot],
                              isems[slot]).wait()

    for slot in range(_R):
        _issue_idx(slot, slot)
    _wait_idx(0)
    pltpu.async_copy(table.at[gidx.at[0]], rows.at[0], gsems[0])

    def _ring(k, _):
        for i in range(_R):
            j = k * _R + i
            b = i
            nb = (i + 1) % _R
            _wait_idx(nb)
            pltpu.async_copy(table.at[gidx.at[nb]], rows.at[nb], gsems[nb])
            pltpu.make_async_copy(table.at[gidx.at[b]], rows.at[b],
                                  gsems[b]).wait()
            pltpu.sync_copy(rows.at[b], accum.at[didx.at[b]], add=True)

            @pl.when(j + _R < _NB)
            def _pf():
                _issue_idx(j + _R, b)
        return None
    lax.fori_loop(0, _NB_MAIN // _R, _ring, None)

    _pb = _NB_MAIN % _R
    pltpu.make_async_copy(table.at[gidx.at[_pb]], rows.at[_pb],
                          gsems[_pb]).wait()
    pltpu.sync_copy(rows.at[_pb], accum.at[didx.at[_pb]], add=True)

    base_t = s * _PER_TILE + _NB * _B
    pltpu.sync_copy(gsrc.at[pl.ds(c * _E + base_t, _TAIL)], gidx2)
    pltpu.sync_copy(dst.at[pl.ds(base_t, _TAIL)], didx2)
    pltpu.async_copy(table.at[gidx2], rows2, tsem).wait()
    pltpu.sync_copy(rows2, accum.at[didx2], add=True)

    plsc.subcore_barrier()

    r0 = s * _RPT
    pltpu.sync_copy(accum.at[pl.ds(r0, _RPT)],
                    out.at[pl.ds(c * _N + r0, _RPT)])

    @pl.when(s == 0)
    def _rtail():
        pltpu.sync_copy(accum.at[pl.ds(_RPT * _TILES, _RTAIL)],
                        out.at[pl.ds(c * _N + _RPT * _TILES, _RTAIL)])


_BLK = 2000
_G = _N // _BLK

_row = lambda i: (i, 0)
_zero = lambda i: (0, 0)
_lo = lambda i: (i, 0)
_hi = lambda i: (_G + i, 0)


def _full(shape):
    return pl.BlockSpec(shape, _zero)


def _tc_init_body(x_ref, ws_ref, bs_ref, wk_ref, bk_ref, se_ref, ke_ref):
    xb = x_ref[...]
    se_ref[...] = xb * ws_ref[...] + bs_ref[...]
    ke_ref[...] = xb * wk_ref[...] + bk_ref[...]


_tc_init = pl.pallas_call(
    _tc_init_body,
    grid=(_G,),
    in_specs=[
        pl.BlockSpec((_BLK, 1), _row),
        _full((1, _H)), _full((1, _H)), _full((1, _H)), _full((1, _H)),
    ],
    out_specs=[pl.BlockSpec((_BLK, _H), _row)] * 2,
    out_shape=[jax.ShapeDtypeStruct((_N, _H), jnp.float32)] * 2,
)


def _tc_layer_body(alo_ref, ahi_ref, e_ref, w_ref, b_ref, o_ref):
    a = jnp.concatenate([alo_ref[...], ahi_ref[...]], axis=1)
    t = a + e_ref[...]
    o_ref[...] = jnp.maximum(
        jnp.dot(t, w_ref[...], preferred_element_type=jnp.float32)
        + b_ref[...], 0.0)


_tc_layer = pl.pallas_call(
    _tc_layer_body,
    grid=(_G,),
    in_specs=[
        pl.BlockSpec((_BLK, _HH), _lo),
        pl.BlockSpec((_BLK, _HH), _hi),
        pl.BlockSpec((_BLK, _H), _row),
        _full((_H, _H)), _full((1, _H)),
    ],
    out_specs=pl.BlockSpec((_BLK, _H), _row),
    out_shape=jax.ShapeDtypeStruct((_N, _H), jnp.float32),
)


def _tc_comb_body(se_ref, ke_ref, x_ref, wso_ref, bso_ref, wko_ref, bko_ref,
                  pd0_ref, v_ref):
    comb = se_ref[...] + ke_ref[...]
    so = jnp.maximum(
        jnp.dot(comb, wso_ref[...], preferred_element_type=jnp.float32)
        + bso_ref[...], 0.0)
    ko = jnp.maximum(
        jnp.dot(comb, wko_ref[...], preferred_element_type=jnp.float32)
        + bko_ref[...], 0.0)
    pd0_ref[...] = (so + ko) * 0.5
    v_ref[...] = comb * x_ref[...]


_tc_comb = pl.pallas_call(
    _tc_comb_body,
    grid=(_G,),
    in_specs=[
        pl.BlockSpec((_BLK, _H), _row),
        pl.BlockSpec((_BLK, _H), _row),
        pl.BlockSpec((_BLK, 1), _row),
        _full((_H, 2)), _full((1, 2)), _full((_H, 2)), _full((1, 2)),
    ],
    out_specs=[
        pl.BlockSpec((_BLK, 2), _row),
        pl.BlockSpec((_BLK, _H), _row),
    ],
    out_shape=[
        jax.ShapeDtypeStruct((_N, 2), jnp.float32),
        jax.ShapeDtypeStruct((_N, _H), jnp.float32),
    ],
)


def _tc_final_body(aslo_ref, ashi_ref, aklo_ref, akhi_ref, pd0_ref,
                   wd_ref, bd_ref, o_ref):
    s1 = jnp.maximum(
        jnp.concatenate([aslo_ref[...], ashi_ref[...]], axis=1), 0.0)
    k1 = jnp.maximum(
        jnp.concatenate([aklo_ref[...], akhi_ref[...]], axis=1), 0.0)
    pd1 = jnp.maximum(
        jnp.dot(s1 + k1, wd_ref[...], preferred_element_type=jnp.float32)
        + bd_ref[...], 0.0)
    o_ref[...] = jnp.concatenate([pd0_ref[...], pd1], axis=1)


_tc_final = pl.pallas_call(
    _tc_final_body,
    grid=(_G,),
    in_specs=[
        pl.BlockSpec((_BLK, _HH), _lo),
        pl.BlockSpec((_BLK, _HH), _hi),
        pl.BlockSpec((_BLK, _HH), _lo),
        pl.BlockSpec((_BLK, _HH), _hi),
        pl.BlockSpec((_BLK, 2), _row),
        _full((_H, 2)), _full((1, 2)),
    ],
    out_specs=pl.BlockSpec((_BLK, 4), _row),
    out_shape=jax.ShapeDtypeStruct((_N, 4), jnp.float32),
)


def kernel(x, source_edge_index, sink_edge_index, W_src_init, b_src_init,
           W_snk_init, b_snk_init, W_layers_src, b_layers_src, W_layers_snk,
           b_layers_snk, W_src_out, b_src_out, W_snk_out, b_snk_out,
           W_dim1_out, b_dim1_out):
    ssrc, sdst = source_edge_index[0], source_edge_index[1]
    ksrc, kdst = sink_edge_index[0], sink_edge_index[1]
    ssrc2 = jnp.concatenate([ssrc * 2, ssrc * 2 + 1])
    ksrc2 = jnp.concatenate([ksrc * 2, ksrc * 2 + 1])

    se, ke = _tc_init(x, W_src_init, b_src_init.reshape(1, _H),
                      W_snk_init, b_snk_init.reshape(1, _H))

    for i in range(W_layers_src.shape[0]):
        aggs = _sc_segsum(ssrc2, sdst, se.reshape(2 * _N, _HH))
        se = _tc_layer(aggs, aggs, se, W_layers_src[i],
                       b_layers_src[i].reshape(1, _H))
        aggk = _sc_segsum(ksrc2, kdst, ke.reshape(2 * _N, _HH))
        ke = _tc_layer(aggk, aggk, ke, W_layers_snk[i],
                       b_layers_snk[i].reshape(1, _H))

    pd0, v = _tc_comb(se, ke, x, W_src_out, b_src_out.reshape(1, 2),
                      W_snk_out, b_snk_out.reshape(1, 2))

    v2 = v.reshape(2 * _N, _HH)
    a_s = _sc_segsum(ssrc2, sdst, v2)
    a_k = _sc_segsum(ksrc2, kdst, v2)

    return _tc_final(a_s, a_s, a_k, a_k, pd0, W_dim1_out,
                     b_dim1_out.reshape(1, 2))


# speedup vs baseline: 22.3505x; 1.3856x over previous
import functools

import jax
import jax.numpy as jnp
from jax import lax
from jax.experimental import pallas as pl
from jax.experimental.pallas import tpu as pltpu
from jax.experimental.pallas import tpu_sc as plsc

_N = 100000
_E = 1600000
_H = 32
_HH = _H // 2
_TILES = 16
_B = 128
_PER_TILE = _E // _TILES
_NB = _PER_TILE // _B
_TAIL = _PER_TILE - _NB * _B
_RPT = 6248
_RTAIL = _N - _RPT * _TILES
_ZCH = 568


_sc_mesh = plsc.VectorSubcoreMesh(core_axis_name="c", subcore_axis_name="s")


_R = 3
_S = 2
_SB = _S * _B
_NS = _PER_TILE // _SB
_LEFT = _PER_TILE - _NS * _SB


@functools.partial(
    pl.kernel,
    out_type=jax.ShapeDtypeStruct((2 * _N, _HH), jnp.float32),
    mesh=_sc_mesh,
    scratch_types=[
        pltpu.VMEM_SHARED((_N, _HH), jnp.float32),
        pltpu.VMEM((_R, _SB), jnp.int32),
        pltpu.VMEM((_R, _S, _B), jnp.int32),
        pltpu.VMEM((_R, _S, _B, _HH), jnp.float32),
        pltpu.VMEM((_B,), jnp.int32),
        pltpu.VMEM((_B,), jnp.int32),
        pltpu.VMEM((_B, _HH), jnp.float32),
        pltpu.VMEM((_TAIL,), jnp.int32),
        pltpu.VMEM((_TAIL,), jnp.int32),
        pltpu.VMEM((_TAIL, _HH), jnp.float32),
        pltpu.VMEM((_ZCH, _HH), jnp.float32),
        [pltpu.SemaphoreType.DMA] * _R,
        [pltpu.SemaphoreType.DMA] * _R,
        [pltpu.SemaphoreType.DMA] * _R,
        pltpu.SemaphoreType.DMA,
    ],
    compiler_params=pltpu.CompilerParams(use_tc_tiling_on_sc=False),
)
def _sc_segsum(gsrc, dst, table, out, accum, gidx, didx, rows,
               gidxL, didxL, rowsL, gidx2, didx2, rows2, zbuf,
               gsems, ssems, isems, tsem):
    c = lax.axis_index("c")
    s = lax.axis_index("s")

    def _zrow(i, _):
        zbuf[i, :] = jnp.zeros((_HH,), jnp.float32)
        return None
    lax.fori_loop(0, _ZCH, _zrow, None)

    def _zchunk(k, _):
        pltpu.sync_copy(zbuf, accum.at[pl.ds(s * _RPT + k * _ZCH, _ZCH)])
        return None
    lax.fori_loop(0, _RPT // _ZCH, _zchunk, None)

    @pl.when(s == 0)
    def _ztail():
        pltpu.sync_copy(zbuf.at[pl.ds(0, _RTAIL)],
                        accum.at[pl.ds(_RPT * _TILES, _RTAIL)])
    plsc.subcore_barrier()

    gbase0 = c * _E + s * _PER_TILE
    dbase0 = s * _PER_TILE

    def _issue_idx(t, slot):
        gb = gbase0 + t * _SB
        db = dbase0 + t * _SB
        pltpu.async_copy(gsrc.at[pl.ds(gb, _SB)], gidx.at[slot], isems[slot])
        for k in range(_S):
            pltpu.async_copy(dst.at[pl.ds(db + k * _B, _B)],
                             didx.at[slot, k], isems[slot])

    def _wait_idx(slot):
        pltpu.make_async_copy(gsrc.at[pl.ds(0, _SB)], gidx.at[slot],
                              isems[slot]).wait()
        for k in range(_S):
            pltpu.make_async_copy(dst.at[pl.ds(0, _B)], didx.at[slot, k],
                                  isems[slot]).wait()

    def _issue_gathers(slot):
        for k in range(_S):
            pltpu.async_copy(table.at[gidx.at[slot, pl.ds(k * _B, _B)]],
                             rows.at[slot, k], gsems[slot])

    def _wait_gathers(slot):
        for k in range(_S):
            pltpu.make_async_copy(table.at[gidx.at[slot, pl.ds(k * _B, _B)]],
                                  rows.at[slot, k], gsems[slot]).wait()

    def _scatters(slot):
        for k in range(_S):
            pltpu.async_copy(rows.at[slot, k], accum.at[didx.at[slot, k]],
                             ssems[slot], add=True)
        for k in range(_S):
            pltpu.make_async_copy(rows.at[slot, k], accum.at[didx.at[slot, k]],
                                  ssems[slot]).wait()

    for slot in range(_R):
        _issue_idx(slot, slot)
    _wait_idx(0)
    _issue_gathers(0)

    def _ring(m, _):
        for i in range(_R):
            t = m * _R + i
            b = i
            nb = (i + 1) % _R

            @pl.when(t + 1 < _NS)
            def _nxt():
                _wait_idx(nb)
                _issue_gathers(nb)
            _wait_gathers(b)
            _scatters(b)

            @pl.when(t + _R < _NS)
            def _pf():
                _issue_idx(t + _R, b)
        return None
    lax.fori_loop(0, _NS // _R, _ring, None)

    base_l = s * _PER_TILE + _NS * _SB
    pltpu.sync_copy(gsrc.at[pl.ds(c * _E + base_l, _B)], gidxL)
    pltpu.sync_copy(dst.at[pl.ds(base_l, _B)], didxL)
    pltpu.async_copy(table.at[gidxL], rowsL, tsem).wait()
    pltpu.sync_copy(rowsL, accum.at[didxL], add=True)

    base_t = base_l + _B
    pltpu.sync_copy(gsrc.at[pl.ds(c * _E + base_t, _TAIL)], gidx2)
    pltpu.sync_copy(dst.at[pl.ds(base_t, _TAIL)], didx2)
    pltpu.async_copy(table.at[gidx2], rows2, tsem).wait()
    pltpu.sync_copy(rows2, accum.at[didx2], add=True)

    plsc.subcore_barrier()

    r0 = s * _RPT
    pltpu.sync_copy(accum.at[pl.ds(r0, _RPT)],
                    out.at[pl.ds(c * _N + r0, _RPT)])

    @pl.when(s == 0)
    def _rtail():
        pltpu.sync_copy(accum.at[pl.ds(_RPT * _TILES, _RTAIL)],
                        out.at[pl.ds(c * _N + _RPT * _TILES, _RTAIL)])


_BLK = 2000
_G = _N // _BLK

_row = lambda i: (i, 0)
_zero = lambda i: (0, 0)
_lo = lambda i: (i, 0)
_hi = lambda i: (_G + i, 0)


def _full(shape):
    return pl.BlockSpec(shape, _zero)


def _tc_init_body(x_ref, ws_ref, bs_ref, wk_ref, bk_ref, se_ref, ke_ref):
    xb = x_ref[...]
    se_ref[...] = xb * ws_ref[...] + bs_ref[...]
    ke_ref[...] = xb * wk_ref[...] + bk_ref[...]


_tc_init = pl.pallas_call(
    _tc_init_body,
    grid=(_G,),
    in_specs=[
        pl.BlockSpec((_BLK, 1), _row),
        _full((1, _H)), _full((1, _H)), _full((1, _H)), _full((1, _H)),
    ],
    out_specs=[pl.BlockSpec((_BLK, _H), _row)] * 2,
    out_shape=[jax.ShapeDtypeStruct((_N, _H), jnp.float32)] * 2,
)


def _tc_layer_body(alo_ref, ahi_ref, e_ref, w_ref, b_ref, o_ref):
    a = jnp.concatenate([alo_ref[...], ahi_ref[...]], axis=1)
    t = a + e_ref[...]
    o_ref[...] = jnp.maximum(
        jnp.dot(t, w_ref[...], preferred_element_type=jnp.float32)
        + b_ref[...], 0.0)


_tc_layer = pl.pallas_call(
    _tc_layer_body,
    grid=(_G,),
    in_specs=[
        pl.BlockSpec((_BLK, _HH), _lo),
        pl.BlockSpec((_BLK, _HH), _hi),
        pl.BlockSpec((_BLK, _H), _row),
        _full((_H, _H)), _full((1, _H)),
    ],
    out_specs=pl.BlockSpec((_BLK, _H), _row),
    out_shape=jax.ShapeDtypeStruct((_N, _H), jnp.float32),
)


def _tc_comb_body(se_ref, ke_ref, x_ref, wso_ref, bso_ref, wko_ref, bko_ref,
                  pd0_ref, v_ref):
    comb = se_ref[...] + ke_ref[...]
    so = jnp.maximum(
        jnp.dot(comb, wso_ref[...], preferred_element_type=jnp.float32)
        + bso_ref[...], 0.0)
    ko = jnp.maximum(
        jnp.dot(comb, wko_ref[...], preferred_element_type=jnp.float32)
        + bko_ref[...], 0.0)
    pd0_ref[...] = (so + ko) * 0.5
    v_ref[...] = comb * x_ref[...]


_tc_comb = pl.pallas_call(
    _tc_comb_body,
    grid=(_G,),
    in_specs=[
        pl.BlockSpec((_BLK, _H), _row),
        pl.BlockSpec((_BLK, _H), _row),
        pl.BlockSpec((_BLK, 1), _row),
        _full((_H, 2)), _full((1, 2)), _full((_H, 2)), _full((1, 2)),
    ],
    out_specs=[
        pl.BlockSpec((_BLK, 2), _row),
        pl.BlockSpec((_BLK, _H), _row),
    ],
    out_shape=[
        jax.ShapeDtypeStruct((_N, 2), jnp.float32),
        jax.ShapeDtypeStruct((_N, _H), jnp.float32),
    ],
)


def _tc_final_body(aslo_ref, ashi_ref, aklo_ref, akhi_ref, pd0_ref,
                   wd_ref, bd_ref, o_ref):
    s1 = jnp.maximum(
        jnp.concatenate([aslo_ref[...], ashi_ref[...]], axis=1), 0.0)
    k1 = jnp.maximum(
        jnp.concatenate([aklo_ref[...], akhi_ref[...]], axis=1), 0.0)
    pd1 = jnp.maximum(
        jnp.dot(s1 + k1, wd_ref[...], preferred_element_type=jnp.float32)
        + bd_ref[...], 0.0)
    o_ref[...] = jnp.concatenate([pd0_ref[...], pd1], axis=1)


_tc_final = pl.pallas_call(
    _tc_final_body,
    grid=(_G,),
    in_specs=[
        pl.BlockSpec((_BLK, _HH), _lo),
        pl.BlockSpec((_BLK, _HH), _hi),
        pl.BlockSpec((_BLK, _HH), _lo),
        pl.BlockSpec((_BLK, _HH), _hi),
        pl.BlockSpec((_BLK, 2), _row),
        _full((_H, 2)), _full((1, 2)),
    ],
    out_specs=pl.BlockSpec((_BLK, 4), _row),
    out_shape=jax.ShapeDtypeStruct((_N, 4), jnp.float32),
)


def kernel(x, source_edge_index, sink_edge_index, W_src_init, b_src_init,
           W_snk_init, b_snk_init, W_layers_src, b_layers_src, W_layers_snk,
           b_layers_snk, W_src_out, b_src_out, W_snk_out, b_snk_out,
           W_dim1_out, b_dim1_out):
    ssrc, sdst = source_edge_index[0], source_edge_index[1]
    ksrc, kdst = sink_edge_index[0], sink_edge_index[1]
    ssrc2 = jnp.concatenate([ssrc * 2, ssrc * 2 + 1])
    ksrc2 = jnp.concatenate([ksrc * 2, ksrc * 2 + 1])

    se, ke = _tc_init(x, W_src_init, b_src_init.reshape(1, _H),
                      W_snk_init, b_snk_init.reshape(1, _H))

    for i in range(W_layers_src.shape[0]):
        aggs = _sc_segsum(ssrc2, sdst, se.reshape(2 * _N, _HH))
        se = _tc_layer(aggs, aggs, se, W_layers_src[i],
                       b_layers_src[i].reshape(1, _H))
        aggk = _sc_segsum(ksrc2, kdst, ke.reshape(2 * _N, _HH))
        ke = _tc_layer(aggk, aggk, ke, W_layers_snk[i],
                       b_layers_snk[i].reshape(1, _H))

    pd0, v = _tc_comb(se, ke, x, W_src_out, b_src_out.reshape(1, 2),
                      W_snk_out, b_snk_out.reshape(1, 2))

    v2 = v.reshape(2 * _N, _HH)
    a_s = _sc_segsum(ssrc2, sdst, v2)
    a_k = _sc_segsum(ksrc2, kdst, v2)

    return _tc_final(a_s, a_s, a_k, a_k, pd0, W_dim1_out,
                     b_dim1_out.reshape(1, 2))
